# Initial kernel scaffold; baseline (speedup 1.0000x reference)
import jax
import jax.numpy as jnp
from jax.experimental import pallas as pl


def kernel(x, edge_index, W1, b1, W2, b2, Wh, bh):
    raise NotImplementedError("write your pallas kernel here")


# speedup vs baseline: 7.6415x; 7.6415x over previous
import functools
import jax
import jax.numpy as jnp
from jax import lax
from jax.experimental import pallas as pl
from jax.experimental.pallas import tpu as pltpu
from jax.experimental.pallas import tpu_sc as plsc

N = 10000
NP = 10240
E = 320000
K = 128
NC, NS = 2, 16
NW = NC * NS
C = 79
E_PAD = NW * C * K
RPT = NP // NS
BM = 1024
GRID = NP // BM

_mesh = plsc.VectorSubcoreMesh(
    core_axis_name="c", subcore_axis_name="s", num_cores=NC, num_subcores=NS)


def _fill(buf, rows, width, value):
    vec = jnp.full((16,), value, jnp.float32)

    def body(r, carry):
        for j in range(width // 16):
            buf[r, pl.ds(j * 16, 16)] = vec
        return carry

    lax.fori_loop(0, rows, body, 0)


def _make_deg():
    @functools.partial(
        pl.kernel,
        out_type=jax.ShapeDtypeStruct((NC, NP, 16), jnp.float32),
        mesh=_mesh,
        scratch_types=[
            pltpu.VMEM((1, K), jnp.int32),
            pltpu.VMEM((K, 16), jnp.float32),
            pltpu.VMEM((K, 16), jnp.float32),
            pltpu.VMEM_SHARED((NP, 16), jnp.float32),
        ],
    )
    def deg_kernel(dst_hbm, out_hbm, didx, ones_b, zero_b, accum):
        c = lax.axis_index("c")
        s = lax.axis_index("s")
        wid = s * NC + c
        base = wid * (C * K)

        _fill(ones_b, K, 16, 1.0)
        _fill(zero_b, K, 16, 0.0)
        for t in range(RPT // K):
            pltpu.sync_copy(zero_b, accum.at[pl.ds(s * RPT + t * K, K)])
        plsc.subcore_barrier()

        def chunk(i, carry):
            off = base + i * K
            pltpu.sync_copy(dst_hbm.at[pl.ds(off, K)], didx.at[0])
            pltpu.sync_copy(ones_b, accum.at[didx.at[0]], add=True)
            return carry

        lax.fori_loop(0, C, chunk, 0)
        plsc.subcore_barrier()
        pltpu.sync_copy(accum.at[pl.ds(s * RPT, RPT)],
                        out_hbm.at[c, pl.ds(s * RPT, RPT)])

    return deg_kernel


def _make_scatter(F):
    @functools.partial(
        pl.kernel,
        out_type=jax.ShapeDtypeStruct((NC, NP, F), jnp.float32),
        mesh=_mesh,
        scratch_types=[
            pltpu.VMEM((1, K), jnp.int32),
            pltpu.VMEM((1, K), jnp.int32),
            pltpu.VMEM((K, F), jnp.float32),
            pltpu.VMEM_SHARED((NP, F), jnp.float32),
            pltpu.SemaphoreType.DMA,
        ],
    )
    def scat_kernel(g_hbm, src_hbm, dst_hbm, out_hbm, sidx, didx, rows, accum, sem):
        c = lax.axis_index("c")
        s = lax.axis_index("s")
        wid = s * NC + c
        base = wid * (C * K)

        _fill(rows, K, F, 0.0)
        for t in range(RPT // K):
            pltpu.sync_copy(rows, accum.at[pl.ds(s * RPT + t * K, K)])
        plsc.subcore_barrier()

        def chunk(i, carry):
            off = base + i * K
            pltpu.sync_copy(src_hbm.at[pl.ds(off, K)], sidx.at[0])
            pltpu.sync_copy(dst_hbm.at[pl.ds(off, K)], didx.at[0])
            pltpu.async_copy(g_hbm.at[sidx.at[0]], rows, sem).wait()
            pltpu.sync_copy(rows, accum.at[didx.at[0]], add=True)
            return carry

        lax.fori_loop(0, C, chunk, 0)
        plsc.subcore_barrier()
        pltpu.sync_copy(accum.at[pl.ds(s * RPT, RPT)],
                        out_hbm.at[c, pl.ds(s * RPT, RPT)])

    return scat_kernel


_deg = _make_deg()
_scat128 = _make_scatter(128)


def _mm_body(x_ref, w_ref, o_ref):
    o_ref[...] = jnp.dot(x_ref[...], w_ref[...],
                         preferred_element_type=jnp.float32)


def _mm(xp, W):
    fin, fout = W.shape
    return pl.pallas_call(
        _mm_body,
        grid=(GRID,),
        in_specs=[pl.BlockSpec((BM, fin), lambda i: (i, 0)),
                  pl.BlockSpec((fin, fout), lambda i: (0, 0))],
        out_specs=pl.BlockSpec((BM, fout), lambda i: (i, 0)),
        out_shape=jax.ShapeDtypeStruct((NP, fout), jnp.float32),
    )(xp, W)


DEGW = 128


def _dinv_of(d_ref):
    return lax.rsqrt(d_ref[0, :, :1] + d_ref[1, :, :1] + 1.0)


def _scale_body(h_ref, d_ref, o_ref):
    o_ref[...] = _dinv_of(d_ref) * h_ref[...]


def _scale(h, degp):
    F = h.shape[1]
    return pl.pallas_call(
        _scale_body,
        grid=(GRID,),
        in_specs=[pl.BlockSpec((BM, F), lambda i: (i, 0)),
                  pl.BlockSpec((NC, BM, DEGW), lambda i: (0, i, 0))],
        out_specs=pl.BlockSpec((BM, F), lambda i: (i, 0)),
        out_shape=jax.ShapeDtypeStruct((NP, F), jnp.float32),
    )(h, degp)


def _mid_body(p_ref, g_ref, d_ref, b_ref, w_ref, o_ref):
    dinv = _dinv_of(d_ref)
    a = dinv * (p_ref[0] + p_ref[1] + g_ref[...]) + b_ref[...]
    r = jnp.maximum(a, 0.0)
    o_ref[...] = dinv * jnp.dot(r, w_ref[...],
                                preferred_element_type=jnp.float32)


def _mid(p1, g1, degp, b1, W2):
    fin, fout = W2.shape
    return pl.pallas_call(
        _mid_body,
        grid=(GRID,),
        in_specs=[pl.BlockSpec((NC, BM, fin), lambda i: (0, i, 0)),
                  pl.BlockSpec((BM, fin), lambda i: (i, 0)),
                  pl.BlockSpec((NC, BM, DEGW), lambda i: (0, i, 0)),
                  pl.BlockSpec((1, fin), lambda i: (0, 0)),
                  pl.BlockSpec((fin, fout), lambda i: (0, 0))],
        out_specs=pl.BlockSpec((BM, fout), lambda i: (i, 0)),
        out_shape=jax.ShapeDtypeStruct((NP, fout), jnp.float32),
    )(p1, g1, degp, b1, W2)


def _head_body(p_ref, g_ref, d_ref, b_ref, w_ref, bh_ref, o_ref):
    dinv = _dinv_of(d_ref)
    a = dinv * (p_ref[0] + p_ref[1] + g_ref[...]) + b_ref[...]
    r = jnp.maximum(a, 0.0)
    o_ref[...] = jnp.dot(r, w_ref[...],
                         preferred_element_type=jnp.float32) + bh_ref[...]


def _head(p2, g2, degp, b2, Wh, bh):
    fin, fout = Wh.shape
    return pl.pallas_call(
        _head_body,
        grid=(GRID,),
        in_specs=[pl.BlockSpec((NC, BM, fin), lambda i: (0, i, 0)),
                  pl.BlockSpec((BM, fin), lambda i: (i, 0)),
                  pl.BlockSpec((NC, BM, DEGW), lambda i: (0, i, 0)),
                  pl.BlockSpec((1, fin), lambda i: (0, 0)),
                  pl.BlockSpec((fin, fout), lambda i: (0, 0)),
                  pl.BlockSpec((1, fout), lambda i: (0, 0))],
        out_specs=pl.BlockSpec((BM, fout), lambda i: (i, 0)),
        out_shape=jax.ShapeDtypeStruct((NP, fout), jnp.float32),
    )(p2, g2, degp, b2, Wh, bh)


def kernel(x, edge_index, W1, b1, W2, b2, Wh, bh):
    pad = jnp.full((E_PAD - E,), N, jnp.int32)
    srcp = jnp.concatenate([edge_index[0], pad])
    dstp = jnp.concatenate([edge_index[1], pad])
    xp = jnp.pad(x, ((0, NP - N), (0, 0)))

    W2p = jnp.pad(W2, ((0, 0), (0, 128 - W2.shape[1])))
    b2p = jnp.pad(b2, (0, 128 - b2.shape[0]))
    Whp = jnp.pad(Wh, ((0, 128 - Wh.shape[0]), (0, 0)))

    ones = jnp.ones((NP, 128), jnp.float32)
    degp = _scat128(ones, srcp, dstp)
    h1 = _mm(xp, W1)
    g1 = _scale(h1, degp)
    p1 = _scat128(g1, srcp, dstp)
    g2 = _mid(p1, g1, degp, b1.reshape(1, -1), W2p)
    p2 = _scat128(g2, srcp, dstp)
    y = _head(p2, g2, degp, b2p.reshape(1, -1), Whp, bh.reshape(1, -1))
    return y[:N]
